# baseline (device time: 31117 ns/iter reference)
import jax
import jax.numpy as jnp
from jax import lax
from jax.experimental import pallas as pl
from jax.experimental.pallas import tpu as pltpu

N_DEV = 16
B, Sq, Skv = 2, 256, 256
HQ, DH = 4, 64
DM = 512
HD = HQ * DH
NCHUNK = N_DEV
CROWS = (B * Sq) // NCHUNK
CPB = Sq // CROWS
BLK = 64
NBLK = Sq // BLK


def kernel(x, Wq, K_ext, V_ext, Wo):
    my = lax.axis_index("i")
    wq_loc = lax.dynamic_slice_in_dim(Wq, my * HD, HD, axis=1)
    wo_loc = lax.dynamic_slice_in_dim(Wo, my * HD, HD, axis=0)

    def body(x_ref, wq_ref, k_ref, v_ref, wo_ref, out_ref,
             acc_ref, rs_ref, rs_send, rs_recv, ag_send, ag_recv):
        my_pos = lax.axis_index("i")

        barrier = pltpu.get_barrier_semaphore()
        for d in range(N_DEV):
            pl.semaphore_signal(barrier, inc=1, device_id=(d,),
                                device_id_type=pl.DeviceIdType.MESH)

        qb = lax.broadcasted_iota(jnp.int32, (Sq, Skv), 0) // BLK
        kb = lax.broadcasted_iota(jnp.int32, (Sq, Skv), 1) // BLK
        mask = kb <= qb

        HROWS = Sq // 2

        def compute_half(b, g):
            kw = (g + 1) * HROWS
            rows = slice(g * HROWS, (g + 1) * HROWS)
            xf = x_ref[b, rows, :]
            q = jnp.dot(xf, wq_ref[:],
                        preferred_element_type=jnp.float32) * 0.125
            ctxs = []
            for h in range(HQ):
                q_bh = q[:, h * DH:(h + 1) * DH]
                k_bh = k_ref[b, 0:kw, h, :]
                v_bh = v_ref[b, 0:kw, h, :]
                s = jnp.dot(q_bh, k_bh.T,
                            preferred_element_type=jnp.float32)
                s = jnp.where(mask[rows, 0:kw], s, -1e9)
                w = jnp.exp(s)
                w = w / jnp.sum(w, axis=-1, keepdims=True)
                ctxs.append(jnp.dot(w, v_bh,
                                    preferred_element_type=jnp.float32))
            ctx = jnp.concatenate(ctxs, axis=1)
            pb = jnp.dot(ctx, wo_ref[:], preferred_element_type=jnp.float32)
            nhc = HROWS // CROWS
            pb_r = pb.astype(jnp.bfloat16).reshape(nhc, CROWS, DM)
            c0 = b * CPB + g * nhc
            for j in range(nhc):
                acc_ref[c0 + j] = pb_r[j]
            return c0, c0 + nhc

        rs_descs = []

        def send_chunks(c_lo, c_hi):
            for c in range(c_lo, c_hi):
                rdma = pltpu.make_async_remote_copy(
                    src_ref=acc_ref.at[c],
                    dst_ref=rs_ref.at[my_pos],
                    send_sem=rs_send.at[c],
                    recv_sem=rs_recv.at[my_pos],
                    device_id=(c,),
                    device_id_type=pl.DeviceIdType.MESH,
                )
                rdma.start()
                rs_descs.append(rdma)

        first = True
        for b in range(B):
            for g in range(2):
                lo, hi = compute_half(b, g)
                if first:
                    pl.semaphore_wait(barrier, N_DEV)
                    first = False
                send_chunks(lo, hi)

        for s in range(N_DEV):
            recv = pltpu.make_async_remote_copy(
                src_ref=rs_ref.at[s], dst_ref=rs_ref.at[s],
                send_sem=rs_send.at[s], recv_sem=rs_recv.at[s],
                device_id=(s,), device_id_type=pl.DeviceIdType.MESH,
            )
            recv.wait_recv()
        total = jnp.sum(rs_ref[:].astype(jnp.float32), axis=0)
        acc_ref[my_pos] = total.astype(jnp.bfloat16)

        for d in range(N_DEV):
            @pl.when(my_pos != d)
            def _():
                rdma = pltpu.make_async_remote_copy(
                    src_ref=acc_ref.at[my_pos],
                    dst_ref=acc_ref.at[my_pos],
                    send_sem=ag_send.at[d],
                    recv_sem=ag_recv.at[my_pos],
                    device_id=(d,),
                    device_id_type=pl.DeviceIdType.MESH,
                )
                rdma.start()

        for s in range(N_DEV):
            @pl.when(my_pos != s)
            def _():
                recv = pltpu.make_async_remote_copy(
                    src_ref=acc_ref.at[s], dst_ref=acc_ref.at[s],
                    send_sem=ag_send.at[s], recv_sem=ag_recv.at[s],
                    device_id=(s,), device_id_type=pl.DeviceIdType.MESH,
                )
                recv.wait_recv()
            b, j = divmod(s, CPB)
            out_ref[b, j * CROWS:(j + 1) * CROWS, :] = (
                acc_ref[s].astype(jnp.float32))

        for rdma in rs_descs:
            rdma.wait_send()
        for d in range(N_DEV):
            @pl.when(my_pos != d)
            def _():
                send = pltpu.make_async_remote_copy(
                    src_ref=acc_ref.at[my_pos], dst_ref=acc_ref.at[my_pos],
                    send_sem=ag_send.at[d], recv_sem=ag_recv.at[my_pos],
                    device_id=(d,), device_id_type=pl.DeviceIdType.MESH,
                )
                send.wait_send()

    return pl.pallas_call(
        body,
        out_shape=jax.ShapeDtypeStruct((B, Sq, DM), jnp.float32),
        in_specs=[pl.BlockSpec(memory_space=pltpu.VMEM)] * 5,
        out_specs=pl.BlockSpec(memory_space=pltpu.VMEM),
        scratch_shapes=[
            pltpu.VMEM((NCHUNK, CROWS, DM), jnp.bfloat16),
            pltpu.VMEM((N_DEV, CROWS, DM), jnp.bfloat16),
            pltpu.SemaphoreType.DMA((N_DEV,)),
            pltpu.SemaphoreType.DMA((N_DEV,)),
            pltpu.SemaphoreType.DMA((N_DEV,)),
            pltpu.SemaphoreType.DMA((N_DEV,)),
        ],
        compiler_params=pltpu.CompilerParams(collective_id=0),
    )(x, wq_loc, K_ext, V_ext, wo_loc)


# device time: 28890 ns/iter; 1.0771x vs baseline; 1.0771x over previous
import jax
import jax.numpy as jnp
from jax import lax
from jax.experimental import pallas as pl
from jax.experimental.pallas import tpu as pltpu

N_DEV = 16
B, Sq, Skv = 2, 256, 256
HQ, DH = 4, 64
DM = 512
HD = HQ * DH
NCHUNK = N_DEV
CROWS = (B * Sq) // NCHUNK
CPB = Sq // CROWS
BLK = 64
NBLK = Sq // BLK


def kernel(x, Wq, K_ext, V_ext, Wo):
    my = lax.axis_index("i")
    wq_loc = lax.dynamic_slice_in_dim(Wq, my * HD, HD, axis=1)
    wo_loc = lax.dynamic_slice_in_dim(Wo, my * HD, HD, axis=0)

    def body(x_ref, wq_ref, k_ref, v_ref, wo_ref, out_ref,
             acc_ref, rs_ref, rs_send, rs_recv, ag_send, ag_recv):
        my_pos = lax.axis_index("i")

        barrier = pltpu.get_barrier_semaphore()
        for d in range(N_DEV):
            pl.semaphore_signal(barrier, inc=1, device_id=(d,),
                                device_id_type=pl.DeviceIdType.MESH)

        qb = lax.broadcasted_iota(jnp.int32, (Sq, Skv), 0) // BLK
        kb = lax.broadcasted_iota(jnp.int32, (Sq, Skv), 1) // BLK
        mask = kb <= qb

        def compute_batch(b):
            xf = x_ref[b]
            q = jnp.dot(xf, wq_ref[:],
                        preferred_element_type=jnp.float32) * 0.125
            ctxs = []
            for h in range(HQ):
                q_bh = q[:, h * DH:(h + 1) * DH]
                k_bh = k_ref[b, :, h, :]
                v_bh = v_ref[b, :, h, :]
                s = jnp.dot(q_bh, k_bh.T,
                            preferred_element_type=jnp.float32)
                w = jnp.exp(jnp.where(mask, s, -1e9))
                denom = jnp.sum(w, axis=-1, keepdims=True)
                ctxs.append(jnp.dot(w, v_bh,
                                    preferred_element_type=jnp.float32)
                            / denom)
            ctx = jnp.concatenate(ctxs, axis=1)
            pb = jnp.dot(ctx, wo_ref[:], preferred_element_type=jnp.float32)
            pb_r = pb.astype(jnp.bfloat16).reshape(CPB, CROWS, DM)
            for j in range(CPB):
                acc_ref[b * CPB + j] = pb_r[j]

        rs_descs = []

        def send_chunks(c_lo, c_hi):
            for c in range(c_lo, c_hi):
                rdma = pltpu.make_async_remote_copy(
                    src_ref=acc_ref.at[c],
                    dst_ref=rs_ref.at[my_pos],
                    send_sem=rs_send.at[c],
                    recv_sem=rs_recv.at[my_pos],
                    device_id=(c,),
                    device_id_type=pl.DeviceIdType.MESH,
                )
                rdma.start()
                rs_descs.append(rdma)

        compute_batch(0)
        pl.semaphore_wait(barrier, N_DEV)
        send_chunks(0, CPB)
        compute_batch(1)
        send_chunks(CPB, NCHUNK)

        for s in range(N_DEV):
            recv = pltpu.make_async_remote_copy(
                src_ref=rs_ref.at[s], dst_ref=rs_ref.at[s],
                send_sem=rs_send.at[s], recv_sem=rs_recv.at[s],
                device_id=(s,), device_id_type=pl.DeviceIdType.MESH,
            )
            recv.wait_recv()
        total = jnp.sum(rs_ref[:].astype(jnp.float32), axis=0)
        acc_ref[my_pos] = total.astype(jnp.bfloat16)

        for d in range(N_DEV):
            @pl.when(my_pos != d)
            def _():
                rdma = pltpu.make_async_remote_copy(
                    src_ref=acc_ref.at[my_pos],
                    dst_ref=acc_ref.at[my_pos],
                    send_sem=ag_send.at[d],
                    recv_sem=ag_recv.at[my_pos],
                    device_id=(d,),
                    device_id_type=pl.DeviceIdType.MESH,
                )
                rdma.start()

        for s in range(N_DEV):
            @pl.when(my_pos != s)
            def _():
                recv = pltpu.make_async_remote_copy(
                    src_ref=acc_ref.at[s], dst_ref=acc_ref.at[s],
                    send_sem=ag_send.at[s], recv_sem=ag_recv.at[s],
                    device_id=(s,), device_id_type=pl.DeviceIdType.MESH,
                )
                recv.wait_recv()
            b, j = divmod(s, CPB)
            out_ref[b, j * CROWS:(j + 1) * CROWS, :] = (
                acc_ref[s].astype(jnp.float32))

        for rdma in rs_descs:
            rdma.wait_send()
        for d in range(N_DEV):
            @pl.when(my_pos != d)
            def _():
                send = pltpu.make_async_remote_copy(
                    src_ref=acc_ref.at[my_pos], dst_ref=acc_ref.at[my_pos],
                    send_sem=ag_send.at[d], recv_sem=ag_recv.at[my_pos],
                    device_id=(d,), device_id_type=pl.DeviceIdType.MESH,
                )
                send.wait_send()

    return pl.pallas_call(
        body,
        out_shape=jax.ShapeDtypeStruct((B, Sq, DM), jnp.float32),
        in_specs=[pl.BlockSpec(memory_space=pltpu.VMEM)] * 5,
        out_specs=pl.BlockSpec(memory_space=pltpu.VMEM),
        scratch_shapes=[
            pltpu.VMEM((NCHUNK, CROWS, DM), jnp.bfloat16),
            pltpu.VMEM((N_DEV, CROWS, DM), jnp.bfloat16),
            pltpu.SemaphoreType.DMA((N_DEV,)),
            pltpu.SemaphoreType.DMA((N_DEV,)),
            pltpu.SemaphoreType.DMA((N_DEV,)),
            pltpu.SemaphoreType.DMA((N_DEV,)),
        ],
        compiler_params=pltpu.CompilerParams(collective_id=0),
    )(x, wq_loc, K_ext, V_ext, wo_loc)
